# baseline (speedup 1.0000x reference)
import functools

import numpy as np
import jax
import jax.numpy as jnp
from jax import lax
from jax.experimental import pallas as pl
from jax.experimental.pallas import tpu as pltpu
from jax.experimental.pallas import tpu_sc as plsc

CUTOFF = 5.0
LOG2 = float(np.log(2.0))


def _ssp(v):
    sp = jnp.log(1.0 + jnp.exp(jnp.minimum(v, 60.0)))
    return jnp.maximum(sp, v) - LOG2


def _cos_cut(r):
    return 0.5 * (jnp.cos(r * (np.pi / CUTOFF)) + 1.0) * (r < CUTOFF).astype(r.dtype)


def _in2f_body(x_ref, wi_ref, y_ref):
    y_ref[:] = jnp.dot(x_ref[:], wi_ref[:], preferred_element_type=jnp.float32)


def _in2f(x2, Wi):
    M, K = x2.shape
    F = Wi.shape[1]
    return pl.pallas_call(
        _in2f_body,
        out_shape=jax.ShapeDtypeStruct((M, F), jnp.float32),
    )(x2, Wi)


def _sc_gather_call(y, gd, gj, gk, NW, CH):
    R, F = y.shape
    n_ch = gd.shape[1]
    E = NW * n_ch * CH
    per_w = n_ch * CH
    mesh = plsc.VectorSubcoreMesh(core_axis_name="c", subcore_axis_name="s")
    out_sds = jax.ShapeDtypeStruct((E, F), jnp.float32)

    rows_t = pltpu.VMEM((CH, F), jnp.float32)
    idx_t = pltpu.VMEM((n_ch, CH), jnp.int32)

    @functools.partial(
        pl.kernel,
        out_type=[out_sds, out_sds, out_sds],
        mesh=mesh,
        scratch_types=(
            [idx_t, idx_t, idx_t]
            + [rows_t] * 12
            + [pltpu.SemaphoreType.DMA] * 8
        ),
    )
    def sc_gather(y_hbm, gd_hbm, gj_hbm, gk_hbm, od_hbm, oj_hbm, ok_hbm,
                  idx_d, idx_j, idx_k,
                  rd0, rj0, rk0, rd1, rj1, rk1, rd2, rj2, rk2, rd3, rj3, rk3,
                  sg0, sg1, sg2, sg3, sw0, sw1, sw2, sw3):
        wid = lax.axis_index("s") * 2 + lax.axis_index("c")
        pltpu.sync_copy(gd_hbm.at[wid], idx_d)
        pltpu.sync_copy(gj_hbm.at[wid], idx_j)
        pltpu.sync_copy(gk_hbm.at[wid], idx_k)
        base = wid * per_w
        idxs = (idx_d, idx_j, idx_k)
        bufs = ((rd0, rj0, rk0), (rd1, rj1, rk1),
                (rd2, rj2, rk2), (rd3, rj3, rk3))
        sgs = (sg0, sg1, sg2, sg3)
        sws = (sw0, sw1, sw2, sw3)
        outs = (od_hbm, oj_hbm, ok_hbm)

        def issue_g(i, p):
            for t in range(3):
                pltpu.async_copy(y_hbm.at[idxs[t].at[i]], bufs[p][t], sgs[p])

        def wait_g(i, p):
            for t in range(3):
                pltpu.make_async_copy(
                    y_hbm.at[idxs[t].at[i]], bufs[p][t], sgs[p]).wait()

        def issue_w(i, p):
            e0 = base + i * CH
            for t in range(3):
                pltpu.async_copy(bufs[p][t], outs[t].at[pl.ds(e0, CH)], sws[p])

        def wait_w(i, p):
            e0 = base + i * CH
            for t in range(3):
                pltpu.make_async_copy(
                    bufs[p][t], outs[t].at[pl.ds(e0, CH)], sws[p]).wait()

        issue_g(0, 0)
        issue_g(1, 1)
        issue_g(2, 2)

        def step(i, p):
            wait_g(i, p)
            @pl.when(jnp.logical_and(i >= 1, i + 3 < n_ch))
            def _():
                wait_w(i - 1, (p + 3) % 4)
            @pl.when(i + 3 < n_ch)
            def _():
                issue_g(i + 3, (p + 3) % 4)
            issue_w(i, p)

        def body(it, carry):
            for q in range(4):
                step(4 * it + q, q)
            return carry

        lax.fori_loop(0, n_ch // 4, body, 0)
        wait_w(n_ch - 4, 0)
        wait_w(n_ch - 3, 1)
        wait_w(n_ch - 2, 2)
        wait_w(n_ch - 1, 3)

    return sc_gather(y, gd, gj, gk)


def _main_body(BA, N, fd_ref, dt_ref, ydg_ref, yj_ref, yk_ref,
               rd_ref, rij_ref, rjk_ref, nm_ref, tm_ref,
               wd1, bd1, wd2, bd2, wt1, bt1, wt2, bt2, wo, bo, wdn, bdn,
               out_ref):
    f32 = jnp.float32
    bf16 = jnp.bfloat16
    F = wd2.shape[1]

    Wd = _ssp(jnp.dot(fd_ref[:].astype(bf16), wd1[:],
                      preferred_element_type=f32) + bd1[:])
    Wd = _ssp(jnp.dot(Wd.astype(bf16), wd2[:], preferred_element_type=f32)
              + bd2[:])
    Wt = _ssp(jnp.dot(dt_ref[:].astype(bf16), wt1[:],
                      preferred_element_type=f32) + bt1[:])
    Wt = _ssp(jnp.dot(Wt.astype(bf16), wt2[:], preferred_element_type=f32)
              + bt2[:])
    cutd = _cos_cut(rd_ref[:]) * nm_ref[:]
    cutt = _cos_cut(rij_ref[:]) * _cos_cut(rjk_ref[:]) * tm_ref[:]
    cd = (ydg_ref[:] * Wd).reshape(BA, N, F) * cutd[:, :, None]
    ct = (yj_ref[:] * yk_ref[:] * Wt).reshape(BA, N, F) \
        * cutt[:, :, None]
    v = jnp.sum(cd + ct, axis=1)
    v = _ssp(jnp.dot(v, wo[:], preferred_element_type=f32) + bo[:])
    out_ref[:] = jnp.dot(v, wdn[:], preferred_element_type=f32) + bdn[:]


def _main(fd2, dt2, ydg, yjg, ykg, rd2, rij2, rjk2, nm2, tm2,
          Wd1, bd1, Wd2, bd2, Wt1, bt1, Wt2, bt2, Wo, bo, Wdense, bdense,
          BA, N, off_blocks, Mc):
    F = Wo.shape[0]
    nsp = Wd1.shape[0]
    dtr = Wt1.shape[0]
    EB = BA * N
    grid = (Mc // BA,)

    def eb_g(i):
        return (off_blocks + i, 0)

    def eb_l(i):
        return (i, 0)

    def full(i):
        return (0, 0)

    edge_spec = lambda K: pl.BlockSpec((EB, K), eb_g)
    gath_spec = lambda K: pl.BlockSpec((EB, K), eb_l)
    atom_spec = pl.BlockSpec((BA, N), eb_g)
    w_spec = lambda s: pl.BlockSpec(s, full)

    return pl.pallas_call(
        functools.partial(_main_body, BA, N),
        grid=grid,
        in_specs=[
            edge_spec(nsp), edge_spec(dtr),
            gath_spec(F), gath_spec(F), gath_spec(F),
            atom_spec, atom_spec, atom_spec, atom_spec, atom_spec,
            w_spec((nsp, F)), w_spec((1, F)), w_spec((F, F)), w_spec((1, F)),
            w_spec((dtr, F)), w_spec((1, F)), w_spec((F, F)), w_spec((1, F)),
            w_spec((F, F)), w_spec((1, F)), w_spec((F, F)), w_spec((1, F)),
        ],
        out_specs=pl.BlockSpec((BA, F), eb_l),
        out_shape=jax.ShapeDtypeStruct((Mc, F), jnp.float32),
    )(fd2, dt2, ydg, yjg, ykg, rd2, rij2, rjk2, nm2, tm2,
      Wd1, bd1, Wd2, bd2, Wt1, bt1, Wt2, bt2, Wo, bo, Wdense, bdense)


def kernel(x, r_double, r_ij, r_jk, neighbors, neighbor_mask, neighbors_j,
           neighbors_k, triple_mask, d_ijk, f_double,
           Wd1, bd1, Wd2, bd2, Wt1, bt1, Wt2, bt2, Wi, Wo, bo, Wdense, bdense):
    B, A, N = neighbors.shape
    nb = x.shape[-1]
    nf = Wi.shape[1]
    nsp = Wd1.shape[0]
    dtr = Wt1.shape[0]
    E = B * A * N
    NW = 32
    CH = 64
    CC = 8
    Ec = E // CC
    Mc = B * A // CC
    n_ch = Ec // (NW * CH)

    y = _in2f(x.reshape(B * A, nb), Wi)

    base = (jnp.arange(B, dtype=jnp.int32) * A)[:, None, None]
    gd = (neighbors.astype(jnp.int32) + base).reshape(CC, NW, n_ch, CH)
    gj = (neighbors_j.astype(jnp.int32) + base).reshape(CC, NW, n_ch, CH)
    gk = (neighbors_k.astype(jnp.int32) + base).reshape(CC, NW, n_ch, CH)

    fd2 = f_double.reshape(E, nsp).astype(jnp.bfloat16)
    dt2 = d_ijk.reshape(E, dtr).astype(jnp.bfloat16)
    rd2 = r_double.reshape(B * A, N)
    rij2 = r_ij.reshape(B * A, N)
    rjk2 = r_jk.reshape(B * A, N)
    nm2 = neighbor_mask.reshape(B * A, N)
    tm2 = triple_mask.reshape(B * A, N)
    w_args = (Wd1.astype(jnp.bfloat16), bd1.reshape(1, nf),
              Wd2.astype(jnp.bfloat16), bd2.reshape(1, nf),
              Wt1.astype(jnp.bfloat16), bt1.reshape(1, nf),
              Wt2.astype(jnp.bfloat16), bt2.reshape(1, nf),
              Wo, bo.reshape(1, nb), Wdense, bdense.reshape(1, nb))

    BA = 64
    outs = []
    for c in range(CC):
        ydg, yjg, ykg = _sc_gather_call(y, gd[c], gj[c], gk[c], NW, CH)
        outs.append(_main(
            fd2, dt2, ydg, yjg, ykg, rd2, rij2, rjk2, nm2, tm2,
            *w_args, BA, N, c * (Mc // BA), Mc))
    return jnp.concatenate(outs, axis=0).reshape(B, A, nb)


# speedup vs baseline: 1.2338x; 1.2338x over previous
import functools

import numpy as np
import jax
import jax.numpy as jnp
from jax import lax
from jax.experimental import pallas as pl
from jax.experimental.pallas import tpu as pltpu
from jax.experimental.pallas import tpu_sc as plsc

CUTOFF = 5.0
LOG2 = float(np.log(2.0))


def _ssp(v):
    sp = jnp.log(1.0 + jnp.exp(jnp.minimum(v, 60.0)))
    return jnp.maximum(sp, v) - LOG2


def _cos_cut(r):
    return 0.5 * (jnp.cos(r * (np.pi / CUTOFF)) + 1.0) * (r < CUTOFF).astype(r.dtype)


def _in2f_body(x_ref, wi_ref, y_ref):
    y_ref[:] = jnp.dot(x_ref[:], wi_ref[:], preferred_element_type=jnp.float32)


def _in2f(x2, Wi):
    M, K = x2.shape
    F = Wi.shape[1]
    return pl.pallas_call(
        _in2f_body,
        out_shape=jax.ShapeDtypeStruct((M, F), jnp.float32),
    )(x2, Wi)


def _sc_gather_call(y, gd, gj, gk, NW, CH):
    R, F = y.shape
    n_ch = gd.shape[1]
    E = NW * n_ch * CH
    per_w = n_ch * CH
    mesh = plsc.VectorSubcoreMesh(core_axis_name="c", subcore_axis_name="s")
    out_sds = jax.ShapeDtypeStruct((E, F), jnp.float32)

    rows_t = pltpu.VMEM((CH, F), jnp.float32)
    idx_t = pltpu.VMEM((n_ch, CH), jnp.int32)

    @functools.partial(
        pl.kernel,
        out_type=[out_sds, out_sds, out_sds],
        mesh=mesh,
        scratch_types=(
            [idx_t, idx_t, idx_t]
            + [rows_t] * 12
            + [pltpu.SemaphoreType.DMA] * 8
        ),
    )
    def sc_gather(y_hbm, gd_hbm, gj_hbm, gk_hbm, od_hbm, oj_hbm, ok_hbm,
                  idx_d, idx_j, idx_k,
                  rd0, rj0, rk0, rd1, rj1, rk1, rd2, rj2, rk2, rd3, rj3, rk3,
                  sg0, sg1, sg2, sg3, sw0, sw1, sw2, sw3):
        wid = lax.axis_index("s") * 2 + lax.axis_index("c")
        pltpu.sync_copy(gd_hbm.at[wid], idx_d)
        pltpu.sync_copy(gj_hbm.at[wid], idx_j)
        pltpu.sync_copy(gk_hbm.at[wid], idx_k)
        base = wid * per_w
        idxs = (idx_d, idx_j, idx_k)
        bufs = ((rd0, rj0, rk0), (rd1, rj1, rk1),
                (rd2, rj2, rk2), (rd3, rj3, rk3))
        sgs = (sg0, sg1, sg2, sg3)
        sws = (sw0, sw1, sw2, sw3)
        outs = (od_hbm, oj_hbm, ok_hbm)

        def issue_g(i, p):
            for t in range(3):
                pltpu.async_copy(y_hbm.at[idxs[t].at[i]], bufs[p][t], sgs[p])

        def wait_g(i, p):
            for t in range(3):
                pltpu.make_async_copy(
                    y_hbm.at[idxs[t].at[i]], bufs[p][t], sgs[p]).wait()

        def issue_w(i, p):
            e0 = base + i * CH
            for t in range(3):
                pltpu.async_copy(bufs[p][t], outs[t].at[pl.ds(e0, CH)], sws[p])

        def wait_w(i, p):
            e0 = base + i * CH
            for t in range(3):
                pltpu.make_async_copy(
                    bufs[p][t], outs[t].at[pl.ds(e0, CH)], sws[p]).wait()

        issue_g(0, 0)
        issue_g(1, 1)
        issue_g(2, 2)

        def step(i, p):
            wait_g(i, p)
            @pl.when(jnp.logical_and(i >= 1, i + 3 < n_ch))
            def _():
                wait_w(i - 1, (p + 3) % 4)
            @pl.when(i + 3 < n_ch)
            def _():
                issue_g(i + 3, (p + 3) % 4)
            issue_w(i, p)

        def body(it, carry):
            for q in range(4):
                step(4 * it + q, q)
            return carry

        lax.fori_loop(0, n_ch // 4, body, 0)
        wait_w(n_ch - 4, 0)
        wait_w(n_ch - 3, 1)
        wait_w(n_ch - 2, 2)
        wait_w(n_ch - 1, 3)

    return sc_gather(y, gd, gj, gk)


def _main_body(BA, N, fd_ref, dt_ref, ydg_ref, yj_ref, yk_ref,
               rd_ref, rij_ref, rjk_ref, nm_ref, tm_ref,
               wd1, bd1, wd2, bd2, wt1, bt1, wt2, bt2, wo, bo, wdn, bdn,
               out_ref):
    f32 = jnp.float32
    bf16 = jnp.bfloat16
    F = wd2.shape[1]

    Wd = _ssp(jnp.dot(fd_ref[:].astype(bf16), wd1[:],
                      preferred_element_type=f32) + bd1[:])
    Wd = _ssp(jnp.dot(Wd.astype(bf16), wd2[:], preferred_element_type=f32)
              + bd2[:])
    Wt = _ssp(jnp.dot(dt_ref[:].astype(bf16), wt1[:],
                      preferred_element_type=f32) + bt1[:])
    Wt = _ssp(jnp.dot(Wt.astype(bf16), wt2[:], preferred_element_type=f32)
              + bt2[:])
    cutd = _cos_cut(rd_ref[:]) * nm_ref[:]
    cutt = _cos_cut(rij_ref[:]) * _cos_cut(rjk_ref[:]) * tm_ref[:]
    cd = (ydg_ref[:] * Wd).reshape(BA, N, F) * cutd[:, :, None]
    ct = (yj_ref[:] * yk_ref[:] * Wt).reshape(BA, N, F) \
        * cutt[:, :, None]
    v = jnp.sum(cd + ct, axis=1)
    v = _ssp(jnp.dot(v, wo[:], preferred_element_type=f32) + bo[:])
    out_ref[:] = jnp.dot(v, wdn[:], preferred_element_type=f32) + bdn[:]


def _main(fd2, dt2, ydg, yjg, ykg, rd2, rij2, rjk2, nm2, tm2,
          Wd1, bd1, Wd2, bd2, Wt1, bt1, Wt2, bt2, Wo, bo, Wdense, bdense,
          BA, N, off_blocks, Mc):
    F = Wo.shape[0]
    nsp = Wd1.shape[0]
    dtr = Wt1.shape[0]
    EB = BA * N
    grid = (Mc // BA,)

    def eb_g(i):
        return (off_blocks + i, 0)

    def eb_l(i):
        return (i, 0)

    def full(i):
        return (0, 0)

    edge_spec = lambda K: pl.BlockSpec((EB, K), eb_g)
    gath_spec = lambda K: pl.BlockSpec((EB, K), eb_l)
    atom_spec = pl.BlockSpec((BA, N), eb_g)
    w_spec = lambda s: pl.BlockSpec(s, full)

    return pl.pallas_call(
        functools.partial(_main_body, BA, N),
        grid=grid,
        in_specs=[
            edge_spec(nsp), edge_spec(dtr),
            gath_spec(F), gath_spec(F), gath_spec(F),
            atom_spec, atom_spec, atom_spec, atom_spec, atom_spec,
            w_spec((nsp, F)), w_spec((1, F)), w_spec((F, F)), w_spec((1, F)),
            w_spec((dtr, F)), w_spec((1, F)), w_spec((F, F)), w_spec((1, F)),
            w_spec((F, F)), w_spec((1, F)), w_spec((F, F)), w_spec((1, F)),
        ],
        out_specs=pl.BlockSpec((BA, F), eb_l),
        out_shape=jax.ShapeDtypeStruct((Mc, F), jnp.float32),
    )(fd2, dt2, ydg, yjg, ykg, rd2, rij2, rjk2, nm2, tm2,
      Wd1, bd1, Wd2, bd2, Wt1, bt1, Wt2, bt2, Wo, bo, Wdense, bdense)


def kernel(x, r_double, r_ij, r_jk, neighbors, neighbor_mask, neighbors_j,
           neighbors_k, triple_mask, d_ijk, f_double,
           Wd1, bd1, Wd2, bd2, Wt1, bt1, Wt2, bt2, Wi, Wo, bo, Wdense, bdense):
    B, A, N = neighbors.shape
    nb = x.shape[-1]
    nf = Wi.shape[1]
    nsp = Wd1.shape[0]
    dtr = Wt1.shape[0]
    E = B * A * N
    NW = 32
    CH = 64
    CC = 2
    Ec = E // CC
    Mc = B * A // CC
    n_ch = Ec // (NW * CH)

    y = _in2f(x.reshape(B * A, nb), Wi)

    base = (jnp.arange(B, dtype=jnp.int32) * A)[:, None, None]
    gd = (neighbors.astype(jnp.int32) + base).reshape(CC, NW, n_ch, CH)
    gj = (neighbors_j.astype(jnp.int32) + base).reshape(CC, NW, n_ch, CH)
    gk = (neighbors_k.astype(jnp.int32) + base).reshape(CC, NW, n_ch, CH)

    fd2 = f_double.reshape(E, nsp).astype(jnp.bfloat16)
    dt2 = d_ijk.reshape(E, dtr).astype(jnp.bfloat16)
    rd2 = r_double.reshape(B * A, N)
    rij2 = r_ij.reshape(B * A, N)
    rjk2 = r_jk.reshape(B * A, N)
    nm2 = neighbor_mask.reshape(B * A, N)
    tm2 = triple_mask.reshape(B * A, N)
    w_args = (Wd1.astype(jnp.bfloat16), bd1.reshape(1, nf),
              Wd2.astype(jnp.bfloat16), bd2.reshape(1, nf),
              Wt1.astype(jnp.bfloat16), bt1.reshape(1, nf),
              Wt2.astype(jnp.bfloat16), bt2.reshape(1, nf),
              Wo, bo.reshape(1, nb), Wdense, bdense.reshape(1, nb))

    BA = 64
    outs = []
    for c in range(CC):
        ydg, yjg, ykg = _sc_gather_call(y, gd[c], gj[c], gk[c], NW, CH)
        outs.append(_main(
            fd2, dt2, ydg, yjg, ykg, rd2, rij2, rjk2, nm2, tm2,
            *w_args, BA, N, c * (Mc // BA), Mc))
    return jnp.concatenate(outs, axis=0).reshape(B, A, nb)
